# Initial kernel scaffold; baseline (speedup 1.0000x reference)
import jax
import jax.numpy as jnp
from jax import lax
from jax.experimental import pallas as pl
from jax.experimental.pallas import tpu as pltpu
from jax.experimental.pallas import tpu_sc as plsc

N_LEVELS = 16
N_FEAT = 2
LOG2_T = 19
TBL = 1 << LOG2_T
N_PTS = 524288
PRIME_I32 = -1640531535
MASK = (1 << LOG2_T) - 1

NC, NS = 2, 16
NW = NC * NS
PW = N_PTS // NW
C = 512
NCHUNK = PW // C
GH = C // 16
GB = C // 8
IDX_BLK = 128


def _body(xs_hbm, ys_hbm, tab_hbm, out_hbm,
          xsv, ysv, wxv, wyv, idxv, rows, outt, sem):
    wid = lax.axis_index("s") * NC + lax.axis_index("c")
    base0 = wid * PW
    iota = lax.iota(jnp.int32, 16)
    half = iota >> 1
    feat = iota & 1
    prime = jnp.int32(PRIME_I32)
    mask = jnp.int32(MASK)

    def chunk_body(ci, _):
        base = base0 + ci * C
        pltpu.sync_copy(xs_hbm.at[pl.ds(base, C)], xsv)
        pltpu.sync_copy(ys_hbm.at[pl.ds(base, C)], ysv)

        def level_body(li, _):
            liv = jnp.full((16,), li, dtype=jnp.int32)
            res_f = (jnp.int32(256) << liv).astype(jnp.float32)
            lbase = liv << LOG2_T
            colv = (liv << 1) + feat

            def hash_body(g, _):
                off = g * 16
                xs = xsv[pl.ds(off, 16)]
                ys = ysv[pl.ds(off, 16)]
                xi = xs * res_f
                yi = ys * res_f
                fx = jnp.floor(xi)
                fy = jnp.floor(yi)
                wxv[pl.ds(off, 16)] = xi - fx
                wyv[pl.ds(off, 16)] = yi - fy
                vx = fx.astype(jnp.int32)
                vy = fy.astype(jnp.int32)
                yp = vy * prime
                yp1 = yp + prime
                vx1 = vx + 1
                idxv[0, pl.ds(off, 16)] = ((vx ^ yp) & mask) + lbase
                idxv[1, pl.ds(off, 16)] = ((vx ^ yp1) & mask) + lbase
                idxv[2, pl.ds(off, 16)] = ((vx1 ^ yp) & mask) + lbase
                idxv[3, pl.ds(off, 16)] = ((vx1 ^ yp1) & mask) + lbase
                return 0

            lax.fori_loop(0, GH, hash_body, 0)

            cps = []
            for c4 in range(4):
                for j in range(C // IDX_BLK):
                    cps.append(pltpu.async_copy(
                        tab_hbm.at[idxv.at[c4, pl.ds(j * IDX_BLK, IDX_BLK)]],
                        rows.at[pl.ds(c4 * C + j * IDX_BLK, IDX_BLK)],
                        sem))
            for cp in cps:
                cp.wait()

            def bil_body(g, _):
                rowbase = half + g * 8
                wx = plsc.load_gather(wxv, [rowbase])
                wy = plsc.load_gather(wyv, [rowbase])
                r0 = plsc.load_gather(rows, [rowbase, feat])
                r1 = plsc.load_gather(rows, [rowbase + C, feat])
                r2 = plsc.load_gather(rows, [rowbase + 2 * C, feat])
                r3 = plsc.load_gather(rows, [rowbase + 3 * C, feat])
                omx = 1.0 - wx
                omy = 1.0 - wy
                c0 = r0 * omx + r2 * wx
                c1 = r1 * omx + r3 * wx
                plsc.store_scatter(outt, [rowbase, colv], c0 * omy + c1 * wy)
                return 0

            lax.fori_loop(0, GB, bil_body, 0)
            return 0

        lax.fori_loop(0, N_LEVELS, level_body, 0)
        pltpu.sync_copy(outt, out_hbm.at[pl.ds(base, C)])
        return 0

    lax.fori_loop(0, NCHUNK, chunk_body, 0)


def kernel(x, tables):
    xs = x[:, 0]
    ys = x[:, 1]
    tab = tables.reshape(N_LEVELS * TBL, N_FEAT)
    mesh = plsc.VectorSubcoreMesh(
        core_axis_name="c", subcore_axis_name="s",
        num_cores=NC, num_subcores=NS)
    f = pl.kernel(
        _body,
        out_type=jax.ShapeDtypeStruct((N_PTS, N_LEVELS * N_FEAT), jnp.float32),
        mesh=mesh,
        scratch_types=[
            pltpu.VMEM((C,), jnp.float32),
            pltpu.VMEM((C,), jnp.float32),
            pltpu.VMEM((C,), jnp.float32),
            pltpu.VMEM((C,), jnp.float32),
            pltpu.VMEM((4, C), jnp.int32),
            pltpu.VMEM((4 * C, N_FEAT), jnp.float32),
            pltpu.VMEM((C, N_LEVELS * N_FEAT), jnp.float32),
            pltpu.SemaphoreType.DMA,
        ],
    )
    return f(xs, ys, tab)


# speedup vs baseline: 21.5811x; 21.5811x over previous
import jax
import jax.numpy as jnp
from jax import lax
from jax.experimental import pallas as pl
from jax.experimental.pallas import tpu as pltpu
from jax.experimental.pallas import tpu_sc as plsc

N_LEVELS = 16
N_FEAT = 2
LOG2_T = 19
TBL = 1 << LOG2_T
N_PTS = 524288
PRIME_I32 = -1640531535
MASK = (1 << LOG2_T) - 1

NC, NS = 2, 16
NW = NC * NS
PW = N_PTS // NW
C = 512
NCHUNK = PW // C
GH = C // 16
GB = C // 8
IDX_BLK = 128
RPC = C // IDX_BLK
PACK = 8


def _body(xs_hbm, ys_hbm, tab_hbm, out_hbm,
          xsv, ysv, wxv, wyv, idxv, selv, rows, outt, sem):
    wid = lax.axis_index("s") * NC + lax.axis_index("c")
    base0 = wid * PW
    iota = lax.iota(jnp.int32, 16)
    half = iota >> 1
    feat = iota & 1
    prime = jnp.int32(PRIME_I32)
    mask = jnp.int32(MASK)

    def chunk_body(ci, _):
        base = base0 + ci * C
        pltpu.sync_copy(xs_hbm.at[pl.ds(base, C)], xsv)
        pltpu.sync_copy(ys_hbm.at[pl.ds(base, C)], ysv)

        for li in range(N_LEVELS):
            res_f = jnp.float32(256.0 * (2.0 ** li))
            lrow = jnp.int32(li << (LOG2_T - 2))
            colv = (2 * li) + feat

            def hash_body(g, _):
                off = g * 16
                xs = xsv[pl.ds(off, 16)]
                ys = ysv[pl.ds(off, 16)]
                xi = xs * res_f
                yi = ys * res_f
                vx = xi.astype(jnp.int32)
                vy = yi.astype(jnp.int32)
                wxv[pl.ds(off, 16)] = xi - vx.astype(jnp.float32)
                wyv[pl.ds(off, 16)] = yi - vy.astype(jnp.float32)
                yp = vy * prime
                yp1 = yp + prime
                vx1 = vx + 1
                h0 = (vx ^ yp) & mask
                h1 = (vx ^ yp1) & mask
                h2 = (vx1 ^ yp) & mask
                h3 = (vx1 ^ yp1) & mask
                row = g >> 3
                col = (g & 7) * 16
                idxv[row, pl.ds(col, 16)] = (h0 >> 2) + lrow
                idxv[row + RPC, pl.ds(col, 16)] = (h1 >> 2) + lrow
                idxv[row + 2 * RPC, pl.ds(col, 16)] = (h2 >> 2) + lrow
                idxv[row + 3 * RPC, pl.ds(col, 16)] = (h3 >> 2) + lrow
                selv[pl.ds(off, 16)] = (h0 & 3) * 2
                selv[pl.ds(C + off, 16)] = (h1 & 3) * 2
                selv[pl.ds(2 * C + off, 16)] = (h2 & 3) * 2
                selv[pl.ds(3 * C + off, 16)] = (h3 & 3) * 2
                return 0

            lax.fori_loop(0, GH, hash_body, 0)

            cps = []
            for r in range(4 * RPC):
                cps.append(pltpu.async_copy(
                    tab_hbm.at[idxv.at[r]],
                    rows.at[pl.ds(r * IDX_BLK, IDX_BLK)],
                    sem))
            for cp in cps:
                cp.wait()

            def bil_body(g, _):
                rowbase = half + g * 8
                wx = plsc.load_gather(wxv, [rowbase])
                wy = plsc.load_gather(wyv, [rowbase])
                s0 = plsc.load_gather(selv, [rowbase]) + feat
                s1 = plsc.load_gather(selv, [rowbase + C]) + feat
                s2 = plsc.load_gather(selv, [rowbase + 2 * C]) + feat
                s3 = plsc.load_gather(selv, [rowbase + 3 * C]) + feat
                r0 = plsc.load_gather(rows, [rowbase, s0])
                r1 = plsc.load_gather(rows, [rowbase + C, s1])
                r2 = plsc.load_gather(rows, [rowbase + 2 * C, s2])
                r3 = plsc.load_gather(rows, [rowbase + 3 * C, s3])
                omx = 1.0 - wx
                omy = 1.0 - wy
                c0 = r0 * omx + r2 * wx
                c1 = r1 * omx + r3 * wx
                plsc.store_scatter(outt, [rowbase, colv], c0 * omy + c1 * wy)
                return 0

            lax.fori_loop(0, GB, bil_body, 0)

        pltpu.sync_copy(outt, out_hbm.at[pl.ds(base, C)])
        return 0

    lax.fori_loop(0, NCHUNK, chunk_body, 0)


def kernel(x, tables):
    xs = x[:, 0]
    ys = x[:, 1]
    tab = tables.reshape(N_LEVELS * TBL * N_FEAT // PACK, PACK)
    mesh = plsc.VectorSubcoreMesh(
        core_axis_name="c", subcore_axis_name="s",
        num_cores=NC, num_subcores=NS)
    f = pl.kernel(
        _body,
        out_type=jax.ShapeDtypeStruct((N_PTS, N_LEVELS * N_FEAT), jnp.float32),
        mesh=mesh,
        compiler_params=pltpu.CompilerParams(
            needs_layout_passes=False, use_tc_tiling_on_sc=False),
        scratch_types=[
            pltpu.VMEM((C,), jnp.float32),
            pltpu.VMEM((C,), jnp.float32),
            pltpu.VMEM((C,), jnp.float32),
            pltpu.VMEM((C,), jnp.float32),
            pltpu.VMEM((4 * RPC, IDX_BLK), jnp.int32),
            pltpu.VMEM((4 * C,), jnp.int32),
            pltpu.VMEM((4 * C, PACK), jnp.float32),
            pltpu.VMEM((C, N_LEVELS * N_FEAT), jnp.float32),
            pltpu.SemaphoreType.DMA,
        ],
    )
    return f(xs, ys, tab)


# speedup vs baseline: 23.4876x; 1.0883x over previous
import jax
import jax.numpy as jnp
from jax import lax
from jax.experimental import pallas as pl
from jax.experimental.pallas import tpu as pltpu
from jax.experimental.pallas import tpu_sc as plsc

N_LEVELS = 16
N_FEAT = 2
LOG2_T = 19
TBL = 1 << LOG2_T
N_PTS = 524288
PRIME_I32 = -1640531535
MASK = (1 << LOG2_T) - 1

NC, NS = 2, 16
NW = NC * NS
PW = N_PTS // NW
C = 512
NCHUNK = PW // C
GH = C // 16
GB = C // 8
IDX_BLK = 128
RPC = C // IDX_BLK
PACK = 8


def _body(xs_hbm, ys_hbm, tab_hbm, out_hbm,
          xsv, ysv, wxv0, wyv0, wxv1, wyv1, idxv0, selv0, idxv1, selv1,
          rows0, rows1, outt, sem0, sem1):
    wid = lax.axis_index("s") * NC + lax.axis_index("c")
    base0 = wid * PW
    iota = lax.iota(jnp.int32, 16)
    half = iota >> 1
    feat = iota & 1
    prime = jnp.int32(PRIME_I32)
    mask = jnp.int32(MASK)
    wxv = (wxv0, wxv1)
    wyv = (wyv0, wyv1)
    idxv = (idxv0, idxv1)
    selv = (selv0, selv1)
    rows = (rows0, rows1)
    sems = (sem0, sem1)

    def hash_level(li, b):
        res_f = jnp.float32(256.0 * (2.0 ** li))
        lrow = jnp.int32(li << (LOG2_T - 2))
        wxb, wyb, idxb, selb = wxv[b], wyv[b], idxv[b], selv[b]

        def hash_body(g, _):
            off = g * 16
            xs = xsv[pl.ds(off, 16)]
            ys = ysv[pl.ds(off, 16)]
            xi = xs * res_f
            yi = ys * res_f
            vx = xi.astype(jnp.int32)
            vy = yi.astype(jnp.int32)
            wxb[pl.ds(off, 16)] = xi - vx.astype(jnp.float32)
            wyb[pl.ds(off, 16)] = yi - vy.astype(jnp.float32)
            yp = vy * prime
            yp1 = yp + prime
            vx1 = vx + 1
            h0 = (vx ^ yp) & mask
            h1 = (vx ^ yp1) & mask
            h2 = (vx1 ^ yp) & mask
            h3 = (vx1 ^ yp1) & mask
            row = g >> 3
            col = (g & 7) * 16
            idxb[row, pl.ds(col, 16)] = (h0 >> 2) + lrow
            idxb[row + RPC, pl.ds(col, 16)] = (h1 >> 2) + lrow
            idxb[row + 2 * RPC, pl.ds(col, 16)] = (h2 >> 2) + lrow
            idxb[row + 3 * RPC, pl.ds(col, 16)] = (h3 >> 2) + lrow
            selb[pl.ds(off, 16)] = (h0 & 3) * 2
            selb[pl.ds(C + off, 16)] = (h1 & 3) * 2
            selb[pl.ds(2 * C + off, 16)] = (h2 & 3) * 2
            selb[pl.ds(3 * C + off, 16)] = (h3 & 3) * 2
            return 0

        lax.fori_loop(0, GH, hash_body, 0)

    def fire_level(b):
        return [pltpu.async_copy(
            tab_hbm.at[idxv[b].at[r]],
            rows[b].at[pl.ds(r * IDX_BLK, IDX_BLK)],
            sems[b]) for r in range(4 * RPC)]

    def bil_level(li, b):
        colv = (2 * li) + feat
        wxb, wyb, selb, rowsb = wxv[b], wyv[b], selv[b], rows[b]

        def bil_body(g, _):
            rowbase = half + g * 8
            wx = plsc.load_gather(wxb, [rowbase])
            wy = plsc.load_gather(wyb, [rowbase])
            s0 = plsc.load_gather(selb, [rowbase]) + feat
            s1 = plsc.load_gather(selb, [rowbase + C]) + feat
            s2 = plsc.load_gather(selb, [rowbase + 2 * C]) + feat
            s3 = plsc.load_gather(selb, [rowbase + 3 * C]) + feat
            r0 = plsc.load_gather(rowsb, [rowbase, s0])
            r1 = plsc.load_gather(rowsb, [rowbase + C, s1])
            r2 = plsc.load_gather(rowsb, [rowbase + 2 * C, s2])
            r3 = plsc.load_gather(rowsb, [rowbase + 3 * C, s3])
            omx = 1.0 - wx
            omy = 1.0 - wy
            c0 = r0 * omx + r2 * wx
            c1 = r1 * omx + r3 * wx
            plsc.store_scatter(outt, [rowbase, colv], c0 * omy + c1 * wy)
            return 0

        lax.fori_loop(0, GB, bil_body, 0)

    def chunk_body(ci, _):
        base = base0 + ci * C
        pltpu.sync_copy(xs_hbm.at[pl.ds(base, C)], xsv)
        pltpu.sync_copy(ys_hbm.at[pl.ds(base, C)], ysv)

        hash_level(0, 0)
        cps = fire_level(0)
        for li in range(N_LEVELS):
            b = li & 1
            nxt_cps = None
            if li + 1 < N_LEVELS:
                hash_level(li + 1, 1 - b)
                nxt_cps = fire_level(1 - b)
            for cp in cps:
                cp.wait()
            bil_level(li, b)
            cps = nxt_cps

        pltpu.sync_copy(outt, out_hbm.at[pl.ds(base, C)])
        return 0

    lax.fori_loop(0, NCHUNK, chunk_body, 0)


def kernel(x, tables):
    xs = x[:, 0]
    ys = x[:, 1]
    tab = tables.reshape(N_LEVELS * TBL * N_FEAT // PACK, PACK)
    mesh = plsc.VectorSubcoreMesh(
        core_axis_name="c", subcore_axis_name="s",
        num_cores=NC, num_subcores=NS)
    f = pl.kernel(
        _body,
        out_type=jax.ShapeDtypeStruct((N_PTS, N_LEVELS * N_FEAT), jnp.float32),
        mesh=mesh,
        compiler_params=pltpu.CompilerParams(
            needs_layout_passes=False, use_tc_tiling_on_sc=False),
        scratch_types=[
            pltpu.VMEM((C,), jnp.float32),
            pltpu.VMEM((C,), jnp.float32),
            pltpu.VMEM((C,), jnp.float32),
            pltpu.VMEM((C,), jnp.float32),
            pltpu.VMEM((C,), jnp.float32),
            pltpu.VMEM((C,), jnp.float32),
            pltpu.VMEM((4 * RPC, IDX_BLK), jnp.int32),
            pltpu.VMEM((4 * C,), jnp.int32),
            pltpu.VMEM((4 * RPC, IDX_BLK), jnp.int32),
            pltpu.VMEM((4 * C,), jnp.int32),
            pltpu.VMEM((4 * C, PACK), jnp.float32),
            pltpu.VMEM((4 * C, PACK), jnp.float32),
            pltpu.VMEM((C, N_LEVELS * N_FEAT), jnp.float32),
            pltpu.SemaphoreType.DMA,
            pltpu.SemaphoreType.DMA,
        ],
    )
    return f(xs, ys, tab)
